# Initial kernel scaffold; baseline (speedup 1.0000x reference)
import jax
import jax.numpy as jnp
from jax.experimental import pallas as pl


def kernel(x, edge_index, batch, W1, b1, W2, b2, W3, b3, W4, b4, W5, b5, W6, b6):
    raise NotImplementedError("write your pallas kernel here")


# speedup vs baseline: 12.1159x; 12.1159x over previous
import functools

import jax
import jax.numpy as jnp
from jax import lax
from jax.experimental import pallas as pl
from jax.experimental.pallas import tpu as pltpu
from jax.experimental.pallas import tpu_sc as plsc

N = 100000
E = 1600000
NUM_GRAPHS = 64
NC, NS, LANES = 2, 16, 16
NW = NC * NS
BATCH = 128
GROUP = 4
PB = 392
GROUPS = PB // GROUP
EPAD = NW * PB * BATCH
EB = EPAD // BATCH
NP = 100352
STRIPE = NP // NS
ZB = STRIPE // BATCH
RBLK = 1000
NBLK = N // RBLK

_mesh = plsc.VectorSubcoreMesh(core_axis_name="c", subcore_axis_name="s")
_sc_params = pltpu.CompilerParams(use_tc_tiling_on_sc=False)


def _sc_deg(col2d):
    scratch = [
        pltpu.VMEM_SHARED((NP, LANES), jnp.float32),
        pltpu.VMEM((GROUP, BATCH), jnp.int32),
        pltpu.VMEM((BATCH, LANES), jnp.float32),
        pltpu.VMEM((BATCH, LANES), jnp.float32),
    ] + [pltpu.SemaphoreType.DMA] * GROUP

    @functools.partial(
        pl.kernel, mesh=_mesh,
        out_type=jax.ShapeDtypeStruct((NC, NP, LANES), jnp.float32),
        scratch_types=scratch, compiler_params=_sc_params,
    )
    def k(col_hbm, out_hbm, acc_sh, coli_v, ones_v, zeros_v, *sems):
        c = lax.axis_index("c")
        s = lax.axis_index("s")
        e1 = jnp.where(lax.iota(jnp.int32, LANES) == 0,
                       jnp.float32(1), jnp.float32(0))
        z16 = jnp.zeros((LANES,), jnp.float32)

        @pl.loop(0, BATCH)
        def _(r):
            ones_v[r, :] = e1
            zeros_v[r, :] = z16

        base_r = s * STRIPE

        @pl.loop(0, ZB)
        def _(z):
            pltpu.sync_copy(zeros_v, acc_sh.at[pl.ds(base_r + z * BATCH, BATCH)])

        plsc.subcore_barrier()
        base_b = (s * NC + c) * PB

        @pl.loop(0, GROUPS)
        def _(g):
            pltpu.sync_copy(col_hbm.at[pl.ds(base_b + g * GROUP, GROUP)], coli_v)
            cps = []
            for j in range(GROUP):
                cps.append(pltpu.async_copy(
                    ones_v, acc_sh.at[coli_v.at[j]], sems[j], add=True))
            for cp in cps:
                cp.wait()

        plsc.subcore_barrier()
        pltpu.sync_copy(acc_sh.at[pl.ds(base_r, STRIPE)],
                        out_hbm.at[c, pl.ds(base_r, STRIPE)])

    return k(col2d)


def _sc_prop(hs_chunks, row2d, col2d):
    P = len(hs_chunks)
    out_type = [jax.ShapeDtypeStruct((NC, NP, LANES), jnp.float32)] * P
    scratch = [
        pltpu.VMEM_SHARED((NP, LANES), jnp.float32),
        pltpu.VMEM((GROUP, BATCH), jnp.int32),
        pltpu.VMEM((GROUP, BATCH), jnp.int32),
        pltpu.VMEM((BATCH, LANES), jnp.float32),
    ] + [pltpu.VMEM((BATCH, LANES), jnp.float32)] * GROUP \
      + [pltpu.SemaphoreType.DMA] * (2 * GROUP)

    @functools.partial(pl.kernel, mesh=_mesh, out_type=out_type,
                       scratch_types=scratch, compiler_params=_sc_params)
    def k(row_hbm, col_hbm, *rest):
        hs_refs = rest[:P]
        out_refs = rest[P:2 * P]
        acc_sh, rowi_v, coli_v, zeros_v = rest[2 * P:2 * P + 4]
        rows_v = rest[2 * P + 4:2 * P + 4 + GROUP]
        gsems = rest[2 * P + 4 + GROUP:2 * P + 4 + 2 * GROUP]
        ssems = rest[2 * P + 4 + 2 * GROUP:]

        c = lax.axis_index("c")
        s = lax.axis_index("s")
        z16 = jnp.zeros((LANES,), jnp.float32)

        @pl.loop(0, BATCH)
        def _(r):
            zeros_v[r, :] = z16

        base_r = s * STRIPE
        base_b = (s * NC + c) * PB

        for p in range(P):
            @pl.loop(0, ZB)
            def _(z):
                pltpu.sync_copy(zeros_v,
                                acc_sh.at[pl.ds(base_r + z * BATCH, BATCH)])

            plsc.subcore_barrier()

            @pl.loop(0, GROUPS)
            def _(g):
                pltpu.sync_copy(
                    row_hbm.at[pl.ds(base_b + g * GROUP, GROUP)], rowi_v)
                pltpu.sync_copy(
                    col_hbm.at[pl.ds(base_b + g * GROUP, GROUP)], coli_v)
                gcps = []
                for j in range(GROUP):
                    gcps.append(pltpu.async_copy(
                        hs_refs[p].at[rowi_v.at[j]], rows_v[j], gsems[j]))
                scps = []
                for j in range(GROUP):
                    gcps[j].wait()
                    scps.append(pltpu.async_copy(
                        rows_v[j], acc_sh.at[coli_v.at[j]], ssems[j],
                        add=True))
                for cp in scps:
                    cp.wait()

            plsc.subcore_barrier()
            pltpu.sync_copy(acc_sh.at[pl.ds(base_r, STRIPE)],
                            out_refs[p].at[c, pl.ds(base_r, STRIPE)])
            plsc.subcore_barrier()

    outs = k(row2d, col2d, *hs_chunks)
    return list(outs) if isinstance(outs, (list, tuple)) else [outs]


def _dinv_of(deg_ref):
    d = deg_ref[0, :, 0:1] + deg_ref[1, :, 0:1] + jnp.float32(1)
    return lax.rsqrt(d)


_deg_spec = pl.BlockSpec((NC, RBLK, LANES), lambda i: (0, i, 0))
_acc_spec = pl.BlockSpec((NC, RBLK, LANES), lambda i: (0, i, 0))
_hs_spec = pl.BlockSpec((RBLK, LANES), lambda i: (i, 0))


def _full_spec(shape):
    return pl.BlockSpec(shape, lambda i: tuple(0 for _ in shape))


def _tc_first(x, deg, W1):
    def body(x_ref, deg_ref, w_ref, o_ref):
        dinv = _dinv_of(deg_ref)
        h = jnp.dot(x_ref[...], w_ref[...],
                    preferred_element_type=jnp.float32,
                    precision=lax.Precision.HIGHEST)
        o_ref[...] = dinv * h

    out = pl.pallas_call(
        body,
        grid=(NBLK,),
        in_specs=[pl.BlockSpec((RBLK, 128), lambda i: (i, 0)),
                  _deg_spec, _full_spec((128, LANES))],
        out_specs=_hs_spec,
        out_shape=jax.ShapeDtypeStruct((N, LANES), jnp.float32),
    )(x, deg, W1)
    return [out]


def _tc_merge(deg, acc_chunks, hs_chunks, b, Wn):
    P_in = len(hs_chunks)
    C_out = Wn.shape[1]
    P_out = C_out // LANES

    def body(*refs):
        deg_ref = refs[0]
        accs = refs[1:1 + P_in]
        hss = refs[1 + P_in:1 + 2 * P_in]
        b_ref, w_ref = refs[1 + 2 * P_in:1 + 2 * P_in + 2]
        outs = refs[1 + 2 * P_in + 2:]
        dinv = _dinv_of(deg_ref)
        acc = jnp.concatenate([a[0] + a[1] for a in accs], axis=1)
        hs = jnp.concatenate([h[...] for h in hss], axis=1)
        a = jnp.maximum(dinv * (acc + hs) + b_ref[...], 0.0)
        h2 = jnp.dot(a, w_ref[...],
                     preferred_element_type=jnp.float32,
                     precision=lax.Precision.HIGHEST)
        for q in range(P_out):
            outs[q][...] = dinv * h2[:, q * LANES:(q + 1) * LANES]

    C_in = P_in * LANES
    outs = pl.pallas_call(
        body,
        grid=(NBLK,),
        in_specs=[_deg_spec] + [_acc_spec] * P_in + [_hs_spec] * P_in
                 + [_full_spec((1, C_in)), _full_spec((C_in, C_out))],
        out_specs=[_hs_spec] * P_out,
        out_shape=[jax.ShapeDtypeStruct((N, LANES), jnp.float32)] * P_out,
    )(deg, *acc_chunks, *hs_chunks, b.reshape(1, C_in), Wn)
    return list(outs)


def _tc_pool(deg, acc_chunks, hs_chunks, b4, batch3d, W5, b5, W6, b6):
    P_in = len(hs_chunks)
    C = P_in * LANES

    def body(*refs):
        deg_ref = refs[0]
        accs = refs[1:1 + P_in]
        hss = refs[1 + P_in:1 + 2 * P_in]
        b4_ref, bt_ref, w5_ref, b5_ref, w6_ref, b6_ref = \
            refs[1 + 2 * P_in:1 + 2 * P_in + 6]
        o_ref = refs[-2]
        gmax = refs[-1]
        i = pl.program_id(0)

        @pl.when(i == 0)
        def _():
            gmax[...] = jnp.full((NUM_GRAPHS, C), -jnp.inf, jnp.float32)

        dinv = _dinv_of(deg_ref)
        acc = jnp.concatenate([a[0] + a[1] for a in accs], axis=1)
        hs = jnp.concatenate([h[...] for h in hss], axis=1)
        a5 = jnp.maximum(dinv * (acc + hs) + b4_ref[...], 0.0)
        bt = bt_ref[0]
        glo = jnp.min(bt)
        ghi = jnp.max(bt)

        def upd(g, carry):
            m = jnp.max(jnp.where(bt == g, a5, -jnp.inf), axis=0)
            cur = gmax[pl.ds(g, 1), :]
            gmax[pl.ds(g, 1), :] = jnp.maximum(cur, m[None, :])
            return carry

        lax.fori_loop(glo, ghi + 1, upd, 0)

        @pl.when(i == NBLK - 1)
        def _():
            gm = gmax[...]
            z = jnp.maximum(
                jnp.dot(gm, w5_ref[...],
                        preferred_element_type=jnp.float32,
                        precision=lax.Precision.HIGHEST) + b5_ref[...], 0.0)
            o_ref[...] = jnp.dot(
                z, w6_ref[...],
                preferred_element_type=jnp.float32,
                precision=lax.Precision.HIGHEST) + b6_ref[...]

    return pl.pallas_call(
        body,
        grid=(NBLK,),
        in_specs=[_deg_spec] + [_acc_spec] * P_in + [_hs_spec] * P_in + [
            _full_spec((1, C)),
            pl.BlockSpec((1, RBLK, 1), lambda i: (i, 0, 0)),
            _full_spec((NUM_GRAPHS, C)),
            _full_spec((1, C)),
            _full_spec((C, 10)),
            _full_spec((1, 10)),
        ],
        out_specs=_full_spec((NUM_GRAPHS, 10)),
        out_shape=jax.ShapeDtypeStruct((NUM_GRAPHS, 10), jnp.float32),
        scratch_shapes=[pltpu.VMEM((NUM_GRAPHS, C), jnp.float32)],
    )(deg, *acc_chunks, *hs_chunks, b4.reshape(1, C), batch3d,
      W5, b5.reshape(1, C), W6, b6.reshape(1, 10))


def kernel(x, edge_index, batch, W1, b1, W2, b2, W3, b3, W4, b4,
           W5, b5, W6, b6):
    pad = EPAD - E
    rowp = jnp.concatenate(
        [edge_index[0], jnp.zeros((pad,), edge_index.dtype)])
    colp = jnp.concatenate(
        [edge_index[1], jnp.full((pad,), N, edge_index.dtype)])
    row2d = rowp.reshape(EB, BATCH)
    col2d = colp.reshape(EB, BATCH)
    batch3d = batch.reshape(NBLK, RBLK, 1)

    deg = _sc_deg(col2d)
    hs = _tc_first(x, deg, W1)
    acc = _sc_prop(hs, row2d, col2d)
    hs = _tc_merge(deg, acc, hs, b1, W2)
    acc = _sc_prop(hs, row2d, col2d)
    hs = _tc_merge(deg, acc, hs, b2, W3)
    acc = _sc_prop(hs, row2d, col2d)
    hs = _tc_merge(deg, acc, hs, b3, W4)
    acc = _sc_prop(hs, row2d, col2d)
    return _tc_pool(deg, acc, hs, b4, batch3d, W5, b5, W6, b6)


# speedup vs baseline: 15.8788x; 1.3106x over previous
import functools

import jax
import jax.numpy as jnp
from jax import lax
from jax.experimental import pallas as pl
from jax.experimental.pallas import tpu as pltpu
from jax.experimental.pallas import tpu_sc as plsc

N = 100000
E = 1600000
NUM_GRAPHS = 64
NC, NS, LANES = 2, 16, 16
NW = NC * NS
BATCH = 128
GROUP = 4
PB = 392
GROUPS = PB // GROUP
EPAD = NW * PB * BATCH
EB = EPAD // BATCH
NP = 100352
STRIPE = NP // NS
ZB = STRIPE // BATCH
RBLK = 1000
NBLK = N // RBLK

_mesh = plsc.VectorSubcoreMesh(core_axis_name="c", subcore_axis_name="s")
_sc_params = pltpu.CompilerParams(use_tc_tiling_on_sc=False)


def _sc_deg(col2d):
    scratch = [
        pltpu.VMEM_SHARED((NP, LANES), jnp.float32),
        pltpu.VMEM((GROUP, BATCH), jnp.int32),
        pltpu.VMEM((BATCH, LANES), jnp.float32),
        pltpu.VMEM((BATCH, LANES), jnp.float32),
    ] + [pltpu.SemaphoreType.DMA] * GROUP

    @functools.partial(
        pl.kernel, mesh=_mesh,
        out_type=jax.ShapeDtypeStruct((NC, NP, LANES), jnp.float32),
        scratch_types=scratch, compiler_params=_sc_params,
    )
    def k(col_hbm, out_hbm, acc_sh, coli_v, ones_v, zeros_v, *sems):
        c = lax.axis_index("c")
        s = lax.axis_index("s")
        e1 = jnp.where(lax.iota(jnp.int32, LANES) == 0,
                       jnp.float32(1), jnp.float32(0))
        z16 = jnp.zeros((LANES,), jnp.float32)

        @pl.loop(0, BATCH)
        def _(r):
            ones_v[r, :] = e1
            zeros_v[r, :] = z16

        base_r = s * STRIPE

        @pl.loop(0, ZB)
        def _(z):
            pltpu.sync_copy(zeros_v, acc_sh.at[pl.ds(base_r + z * BATCH, BATCH)])

        plsc.subcore_barrier()
        base_b = (s * NC + c) * PB

        @pl.loop(0, GROUPS)
        def _(g):
            pltpu.sync_copy(col_hbm.at[pl.ds(base_b + g * GROUP, GROUP)], coli_v)
            cps = []
            for j in range(GROUP):
                cps.append(pltpu.async_copy(
                    ones_v, acc_sh.at[coli_v.at[j]], sems[j], add=True))
            for cp in cps:
                cp.wait()

        plsc.subcore_barrier()
        pltpu.sync_copy(acc_sh.at[pl.ds(base_r, STRIPE)],
                        out_hbm.at[c, pl.ds(base_r, STRIPE)])

    return k(col2d)


PAIRS = (GROUPS - 2) // 2


def _sc_prop(hs_chunks, row2d, col2d):
    P = len(hs_chunks)
    out_type = [jax.ShapeDtypeStruct((NC, NP, LANES), jnp.float32)] * P
    scratch = [
        pltpu.VMEM_SHARED((NP, LANES), jnp.float32),
        pltpu.VMEM((GROUP, BATCH), jnp.int32),
        pltpu.VMEM((GROUP, BATCH), jnp.int32),
        pltpu.VMEM((GROUP, BATCH), jnp.int32),
        pltpu.VMEM((GROUP, BATCH), jnp.int32),
        pltpu.VMEM((BATCH, LANES), jnp.float32),
    ] + [pltpu.VMEM((BATCH, LANES), jnp.float32)] * (2 * GROUP) \
      + [pltpu.SemaphoreType.DMA] * (2 * GROUP) \
      + [pltpu.SemaphoreType.DMA] * 2 \
      + [pltpu.SemaphoreType.DMA] * 2

    @functools.partial(pl.kernel, mesh=_mesh, out_type=out_type,
                       scratch_types=scratch, compiler_params=_sc_params)
    def k(row_hbm, col_hbm, *rest):
        hs_refs = rest[:P]
        out_refs = rest[P:2 * P]
        base = 2 * P
        acc_sh = rest[base]
        rowi = [rest[base + 1], rest[base + 3]]
        coli = [rest[base + 2], rest[base + 4]]
        zeros_v = rest[base + 5]
        rows = [rest[base + 6:base + 6 + GROUP],
                rest[base + 6 + GROUP:base + 6 + 2 * GROUP]]
        gs = base + 6 + 2 * GROUP
        gsems = [rest[gs:gs + GROUP], rest[gs + GROUP:gs + 2 * GROUP]]
        ssems = rest[gs + 2 * GROUP:gs + 2 * GROUP + 2]
        isems = rest[gs + 2 * GROUP + 2:gs + 2 * GROUP + 4]

        c = lax.axis_index("c")
        s = lax.axis_index("s")
        z16 = jnp.zeros((LANES,), jnp.float32)

        @pl.loop(0, BATCH)
        def _(r):
            zeros_v[r, :] = z16

        base_r = s * STRIPE
        base_b = (s * NC + c) * PB

        def idx_fire(b, g):
            pltpu.async_copy(
                row_hbm.at[pl.ds(base_b + g * GROUP, GROUP)], rowi[b],
                isems[b])
            pltpu.async_copy(
                col_hbm.at[pl.ds(base_b + g * GROUP, GROUP)], coli[b],
                isems[b])

        def idx_wait(b):
            pltpu.make_async_copy(
                row_hbm.at[pl.ds(0, GROUP)], rowi[b], isems[b]).wait()
            pltpu.make_async_copy(
                col_hbm.at[pl.ds(0, GROUP)], coli[b], isems[b]).wait()

        def gather_fire(b, p):
            for j in range(GROUP):
                pltpu.async_copy(hs_refs[p].at[rowi[b].at[j]], rows[b][j],
                                 gsems[b][j])

        def gather_wait(b, j, p):
            pltpu.make_async_copy(
                hs_refs[p].at[pl.ds(0, BATCH)], rows[b][j],
                gsems[b][j]).wait()

        def section(g, b, p, prefetch, next_gathers):
            if next_gathers:
                idx_wait(1 - b)
            scps = []
            for j in range(GROUP):
                gather_wait(b, j, p)
                scps.append(pltpu.async_copy(
                    rows[b][j], acc_sh.at[coli[b].at[j]], ssems[b],
                    add=True))
            if next_gathers:
                gather_fire(1 - b, p)
            for cp in scps:
                cp.wait()
            if prefetch:
                idx_fire(b, g + 2)

        for p in range(P):
            @pl.loop(0, ZB)
            def _(z):
                pltpu.sync_copy(zeros_v,
                                acc_sh.at[pl.ds(base_r + z * BATCH, BATCH)])

            plsc.subcore_barrier()

            idx_fire(0, 0)
            idx_wait(0)
            gather_fire(0, p)
            idx_fire(1, 1)

            @pl.loop(0, PAIRS)
            def _(t):
                section(2 * t, 0, p, prefetch=True, next_gathers=True)
                section(2 * t + 1, 1, p, prefetch=True, next_gathers=True)

            section(GROUPS - 2, 0, p, prefetch=False, next_gathers=True)
            section(GROUPS - 1, 1, p, prefetch=False, next_gathers=False)

            plsc.subcore_barrier()
            pltpu.sync_copy(acc_sh.at[pl.ds(base_r, STRIPE)],
                            out_refs[p].at[c, pl.ds(base_r, STRIPE)])
            plsc.subcore_barrier()

    outs = k(row2d, col2d, *hs_chunks)
    return list(outs) if isinstance(outs, (list, tuple)) else [outs]


def _dinv_of(deg_ref):
    d = deg_ref[0, :, 0:1] + deg_ref[1, :, 0:1] + jnp.float32(1)
    return lax.rsqrt(d)


_deg_spec = pl.BlockSpec((NC, RBLK, LANES), lambda i: (0, i, 0))
_acc_spec = pl.BlockSpec((NC, RBLK, LANES), lambda i: (0, i, 0))
_hs_spec = pl.BlockSpec((RBLK, LANES), lambda i: (i, 0))


def _full_spec(shape):
    return pl.BlockSpec(shape, lambda i: tuple(0 for _ in shape))


def _tc_first(x, deg, W1):
    def body(x_ref, deg_ref, w_ref, o_ref):
        dinv = _dinv_of(deg_ref)
        h = jnp.dot(x_ref[...], w_ref[...],
                    preferred_element_type=jnp.float32,
                    precision=lax.Precision.HIGHEST)
        o_ref[...] = dinv * h

    out = pl.pallas_call(
        body,
        grid=(NBLK,),
        in_specs=[pl.BlockSpec((RBLK, 128), lambda i: (i, 0)),
                  _deg_spec, _full_spec((128, LANES))],
        out_specs=_hs_spec,
        out_shape=jax.ShapeDtypeStruct((N, LANES), jnp.float32),
    )(x, deg, W1)
    return [out]


def _tc_merge(deg, acc_chunks, hs_chunks, b, Wn):
    P_in = len(hs_chunks)
    C_out = Wn.shape[1]
    P_out = C_out // LANES

    def body(*refs):
        deg_ref = refs[0]
        accs = refs[1:1 + P_in]
        hss = refs[1 + P_in:1 + 2 * P_in]
        b_ref, w_ref = refs[1 + 2 * P_in:1 + 2 * P_in + 2]
        outs = refs[1 + 2 * P_in + 2:]
        dinv = _dinv_of(deg_ref)
        acc = jnp.concatenate([a[0] + a[1] for a in accs], axis=1)
        hs = jnp.concatenate([h[...] for h in hss], axis=1)
        a = jnp.maximum(dinv * (acc + hs) + b_ref[...], 0.0)
        h2 = jnp.dot(a, w_ref[...],
                     preferred_element_type=jnp.float32,
                     precision=lax.Precision.HIGHEST)
        for q in range(P_out):
            outs[q][...] = dinv * h2[:, q * LANES:(q + 1) * LANES]

    C_in = P_in * LANES
    outs = pl.pallas_call(
        body,
        grid=(NBLK,),
        in_specs=[_deg_spec] + [_acc_spec] * P_in + [_hs_spec] * P_in
                 + [_full_spec((1, C_in)), _full_spec((C_in, C_out))],
        out_specs=[_hs_spec] * P_out,
        out_shape=[jax.ShapeDtypeStruct((N, LANES), jnp.float32)] * P_out,
    )(deg, *acc_chunks, *hs_chunks, b.reshape(1, C_in), Wn)
    return list(outs)


def _tc_pool(deg, acc_chunks, hs_chunks, b4, batch3d, W5, b5, W6, b6):
    P_in = len(hs_chunks)
    C = P_in * LANES

    def body(*refs):
        deg_ref = refs[0]
        accs = refs[1:1 + P_in]
        hss = refs[1 + P_in:1 + 2 * P_in]
        b4_ref, bt_ref, w5_ref, b5_ref, w6_ref, b6_ref = \
            refs[1 + 2 * P_in:1 + 2 * P_in + 6]
        o_ref = refs[-2]
        gmax = refs[-1]
        i = pl.program_id(0)

        @pl.when(i == 0)
        def _():
            gmax[...] = jnp.full((NUM_GRAPHS, C), -jnp.inf, jnp.float32)

        dinv = _dinv_of(deg_ref)
        acc = jnp.concatenate([a[0] + a[1] for a in accs], axis=1)
        hs = jnp.concatenate([h[...] for h in hss], axis=1)
        a5 = jnp.maximum(dinv * (acc + hs) + b4_ref[...], 0.0)
        bt = bt_ref[0]
        glo = jnp.min(bt)
        ghi = jnp.max(bt)

        def upd(g, carry):
            m = jnp.max(jnp.where(bt == g, a5, -jnp.inf), axis=0)
            cur = gmax[pl.ds(g, 1), :]
            gmax[pl.ds(g, 1), :] = jnp.maximum(cur, m[None, :])
            return carry

        lax.fori_loop(glo, ghi + 1, upd, 0)

        @pl.when(i == NBLK - 1)
        def _():
            gm = gmax[...]
            z = jnp.maximum(
                jnp.dot(gm, w5_ref[...],
                        preferred_element_type=jnp.float32,
                        precision=lax.Precision.HIGHEST) + b5_ref[...], 0.0)
            o_ref[...] = jnp.dot(
                z, w6_ref[...],
                preferred_element_type=jnp.float32,
                precision=lax.Precision.HIGHEST) + b6_ref[...]

    return pl.pallas_call(
        body,
        grid=(NBLK,),
        in_specs=[_deg_spec] + [_acc_spec] * P_in + [_hs_spec] * P_in + [
            _full_spec((1, C)),
            pl.BlockSpec((1, RBLK, 1), lambda i: (i, 0, 0)),
            _full_spec((NUM_GRAPHS, C)),
            _full_spec((1, C)),
            _full_spec((C, 10)),
            _full_spec((1, 10)),
        ],
        out_specs=_full_spec((NUM_GRAPHS, 10)),
        out_shape=jax.ShapeDtypeStruct((NUM_GRAPHS, 10), jnp.float32),
        scratch_shapes=[pltpu.VMEM((NUM_GRAPHS, C), jnp.float32)],
    )(deg, *acc_chunks, *hs_chunks, b4.reshape(1, C), batch3d,
      W5, b5.reshape(1, C), W6, b6.reshape(1, 10))


def kernel(x, edge_index, batch, W1, b1, W2, b2, W3, b3, W4, b4,
           W5, b5, W6, b6):
    pad = EPAD - E
    rowp = jnp.concatenate(
        [edge_index[0], jnp.zeros((pad,), edge_index.dtype)])
    colp = jnp.concatenate(
        [edge_index[1], jnp.full((pad,), N, edge_index.dtype)])
    row2d = rowp.reshape(EB, BATCH)
    col2d = colp.reshape(EB, BATCH)
    batch3d = batch.reshape(NBLK, RBLK, 1)

    deg = _sc_deg(col2d)
    hs = _tc_first(x, deg, W1)
    acc = _sc_prop(hs, row2d, col2d)
    hs = _tc_merge(deg, acc, hs, b1, W2)
    acc = _sc_prop(hs, row2d, col2d)
    hs = _tc_merge(deg, acc, hs, b2, W3)
    acc = _sc_prop(hs, row2d, col2d)
    hs = _tc_merge(deg, acc, hs, b3, W4)
    acc = _sc_prop(hs, row2d, col2d)
    return _tc_pool(deg, acc, hs, b4, batch3d, W5, b5, W6, b6)


# speedup vs baseline: 16.6080x; 1.0459x over previous
import functools

import jax
import jax.numpy as jnp
from jax import lax
from jax.experimental import pallas as pl
from jax.experimental.pallas import tpu as pltpu
from jax.experimental.pallas import tpu_sc as plsc

N = 100000
E = 1600000
NUM_GRAPHS = 64
NC, NS, LANES = 2, 16, 16
NW = NC * NS
BATCH = 128
GROUP = 4
PB = 392
GROUPS = PB // GROUP
EPAD = NW * PB * BATCH
EB = EPAD // BATCH
NP = 100352
STRIPE = NP // NS
ZB = STRIPE // BATCH
RBLK = 2000
NBLK = N // RBLK

_mesh = plsc.VectorSubcoreMesh(core_axis_name="c", subcore_axis_name="s")
_sc_params = pltpu.CompilerParams(use_tc_tiling_on_sc=False)


def _sc_deg(col2d):
    scratch = [
        pltpu.VMEM_SHARED((NP, LANES), jnp.float32),
        pltpu.VMEM((GROUP, BATCH), jnp.int32),
        pltpu.VMEM((BATCH, LANES), jnp.float32),
        pltpu.VMEM((BATCH, LANES), jnp.float32),
    ] + [pltpu.SemaphoreType.DMA] * GROUP

    @functools.partial(
        pl.kernel, mesh=_mesh,
        out_type=jax.ShapeDtypeStruct((NC, NP, LANES), jnp.float32),
        scratch_types=scratch, compiler_params=_sc_params,
    )
    def k(col_hbm, out_hbm, acc_sh, coli_v, ones_v, zeros_v, *sems):
        c = lax.axis_index("c")
        s = lax.axis_index("s")
        e1 = jnp.where(lax.iota(jnp.int32, LANES) == 0,
                       jnp.float32(1), jnp.float32(0))
        z16 = jnp.zeros((LANES,), jnp.float32)

        @pl.loop(0, BATCH)
        def _(r):
            ones_v[r, :] = e1
            zeros_v[r, :] = z16

        base_r = s * STRIPE

        @pl.loop(0, ZB)
        def _(z):
            pltpu.sync_copy(zeros_v, acc_sh.at[pl.ds(base_r + z * BATCH, BATCH)])

        plsc.subcore_barrier()
        base_b = (s * NC + c) * PB

        @pl.loop(0, GROUPS)
        def _(g):
            pltpu.sync_copy(col_hbm.at[pl.ds(base_b + g * GROUP, GROUP)], coli_v)
            cps = []
            for j in range(GROUP):
                cps.append(pltpu.async_copy(
                    ones_v, acc_sh.at[coli_v.at[j]], sems[j], add=True))
            for cp in cps:
                cp.wait()

        plsc.subcore_barrier()
        pltpu.sync_copy(acc_sh.at[pl.ds(base_r, STRIPE)],
                        out_hbm.at[c, pl.ds(base_r, STRIPE)])

    return k(col2d)


PAIRS = (GROUPS - 2) // 2


def _sc_prop(hs_chunks, row2d, col2d):
    P = len(hs_chunks)
    out_type = [jax.ShapeDtypeStruct((NC, NP, LANES), jnp.float32)] * P
    scratch = [
        pltpu.VMEM_SHARED((NP, LANES), jnp.float32),
        pltpu.VMEM((GROUP, BATCH), jnp.int32),
        pltpu.VMEM((GROUP, BATCH), jnp.int32),
        pltpu.VMEM((GROUP, BATCH), jnp.int32),
        pltpu.VMEM((GROUP, BATCH), jnp.int32),
        pltpu.VMEM((BATCH, LANES), jnp.float32),
    ] + [pltpu.VMEM((BATCH, LANES), jnp.float32)] * (2 * GROUP) \
      + [pltpu.SemaphoreType.DMA] * (2 * GROUP) \
      + [pltpu.SemaphoreType.DMA] * 2 \
      + [pltpu.SemaphoreType.DMA] * 2

    @functools.partial(pl.kernel, mesh=_mesh, out_type=out_type,
                       scratch_types=scratch, compiler_params=_sc_params)
    def k(row_hbm, col_hbm, *rest):
        hs_refs = rest[:P]
        out_refs = rest[P:2 * P]
        base = 2 * P
        acc_sh = rest[base]
        rowi = [rest[base + 1], rest[base + 3]]
        coli = [rest[base + 2], rest[base + 4]]
        zeros_v = rest[base + 5]
        rows = [rest[base + 6:base + 6 + GROUP],
                rest[base + 6 + GROUP:base + 6 + 2 * GROUP]]
        gs = base + 6 + 2 * GROUP
        gsems = [rest[gs:gs + GROUP], rest[gs + GROUP:gs + 2 * GROUP]]
        ssems = rest[gs + 2 * GROUP:gs + 2 * GROUP + 2]
        isems = rest[gs + 2 * GROUP + 2:gs + 2 * GROUP + 4]

        c = lax.axis_index("c")
        s = lax.axis_index("s")
        z16 = jnp.zeros((LANES,), jnp.float32)

        @pl.loop(0, BATCH)
        def _(r):
            zeros_v[r, :] = z16

        base_r = s * STRIPE
        base_b = (s * NC + c) * PB

        def idx_fire(b, g):
            pltpu.async_copy(
                row_hbm.at[pl.ds(base_b + g * GROUP, GROUP)], rowi[b],
                isems[b])
            pltpu.async_copy(
                col_hbm.at[pl.ds(base_b + g * GROUP, GROUP)], coli[b],
                isems[b])

        def idx_wait(b):
            pltpu.make_async_copy(
                row_hbm.at[pl.ds(0, GROUP)], rowi[b], isems[b]).wait()
            pltpu.make_async_copy(
                col_hbm.at[pl.ds(0, GROUP)], coli[b], isems[b]).wait()

        def gather_fire(b, p):
            for j in range(GROUP):
                pltpu.async_copy(hs_refs[p].at[rowi[b].at[j]], rows[b][j],
                                 gsems[b][j])

        def gather_wait(b, j, p):
            pltpu.make_async_copy(
                hs_refs[p].at[pl.ds(0, BATCH)], rows[b][j],
                gsems[b][j]).wait()

        def section(g, b, p, prefetch, next_gathers):
            if next_gathers:
                idx_wait(1 - b)
            scps = []
            for j in range(GROUP):
                gather_wait(b, j, p)
                scps.append(pltpu.async_copy(
                    rows[b][j], acc_sh.at[coli[b].at[j]], ssems[b],
                    add=True))
            if next_gathers:
                gather_fire(1 - b, p)
            for cp in scps:
                cp.wait()
            if prefetch:
                idx_fire(b, g + 2)

        for p in range(P):
            @pl.loop(0, ZB)
            def _(z):
                pltpu.sync_copy(zeros_v,
                                acc_sh.at[pl.ds(base_r + z * BATCH, BATCH)])

            plsc.subcore_barrier()

            idx_fire(0, 0)
            idx_wait(0)
            gather_fire(0, p)
            idx_fire(1, 1)

            @pl.loop(0, PAIRS)
            def _(t):
                section(2 * t, 0, p, prefetch=True, next_gathers=True)
                section(2 * t + 1, 1, p, prefetch=True, next_gathers=True)

            section(GROUPS - 2, 0, p, prefetch=False, next_gathers=True)
            section(GROUPS - 1, 1, p, prefetch=False, next_gathers=False)

            plsc.subcore_barrier()
            pltpu.sync_copy(acc_sh.at[pl.ds(base_r, STRIPE)],
                            out_refs[p].at[c, pl.ds(base_r, STRIPE)])
            plsc.subcore_barrier()

    outs = k(row2d, col2d, *hs_chunks)
    return list(outs) if isinstance(outs, (list, tuple)) else [outs]


def _dinv_of(deg_ref):
    d = deg_ref[0, :, 0:1] + deg_ref[1, :, 0:1] + jnp.float32(1)
    return lax.rsqrt(d)


_deg_spec = pl.BlockSpec((NC, RBLK, LANES), lambda i: (0, i, 0))
_acc_spec = pl.BlockSpec((NC, RBLK, LANES), lambda i: (0, i, 0))
_hs_spec = pl.BlockSpec((RBLK, LANES), lambda i: (i, 0))


def _full_spec(shape):
    return pl.BlockSpec(shape, lambda i: tuple(0 for _ in shape))


def _tc_first(x, deg, W1):
    def body(x_ref, deg_ref, w_ref, o_ref):
        dinv = _dinv_of(deg_ref)
        h = jnp.dot(x_ref[...], w_ref[...],
                    preferred_element_type=jnp.float32,
                    precision=lax.Precision.HIGHEST)
        o_ref[...] = dinv * h

    out = pl.pallas_call(
        body,
        grid=(NBLK,),
        in_specs=[pl.BlockSpec((RBLK, 128), lambda i: (i, 0)),
                  _deg_spec, _full_spec((128, LANES))],
        out_specs=_hs_spec,
        out_shape=jax.ShapeDtypeStruct((N, LANES), jnp.float32),
    )(x, deg, W1)
    return [out]


def _tc_merge(deg, acc_chunks, hs_chunks, b, Wn):
    P_in = len(hs_chunks)
    C_out = Wn.shape[1]
    P_out = C_out // LANES

    def body(*refs):
        deg_ref = refs[0]
        accs = refs[1:1 + P_in]
        hss = refs[1 + P_in:1 + 2 * P_in]
        b_ref, w_ref = refs[1 + 2 * P_in:1 + 2 * P_in + 2]
        outs = refs[1 + 2 * P_in + 2:]
        dinv = _dinv_of(deg_ref)
        acc = jnp.concatenate([a[0] + a[1] for a in accs], axis=1)
        hs = jnp.concatenate([h[...] for h in hss], axis=1)
        a = jnp.maximum(dinv * (acc + hs) + b_ref[...], 0.0)
        h2 = jnp.dot(a, w_ref[...],
                     preferred_element_type=jnp.float32,
                     precision=lax.Precision.HIGHEST)
        for q in range(P_out):
            outs[q][...] = dinv * h2[:, q * LANES:(q + 1) * LANES]

    C_in = P_in * LANES
    outs = pl.pallas_call(
        body,
        grid=(NBLK,),
        in_specs=[_deg_spec] + [_acc_spec] * P_in + [_hs_spec] * P_in
                 + [_full_spec((1, C_in)), _full_spec((C_in, C_out))],
        out_specs=[_hs_spec] * P_out,
        out_shape=[jax.ShapeDtypeStruct((N, LANES), jnp.float32)] * P_out,
    )(deg, *acc_chunks, *hs_chunks, b.reshape(1, C_in), Wn)
    return list(outs)


def _tc_pool(deg, acc_chunks, hs_chunks, b4, batch3d, W5, b5, W6, b6):
    P_in = len(hs_chunks)
    C = P_in * LANES

    def body(*refs):
        deg_ref = refs[0]
        accs = refs[1:1 + P_in]
        hss = refs[1 + P_in:1 + 2 * P_in]
        b4_ref, bt_ref, w5_ref, b5_ref, w6_ref, b6_ref = \
            refs[1 + 2 * P_in:1 + 2 * P_in + 6]
        o_ref = refs[-2]
        gmax = refs[-1]
        i = pl.program_id(0)

        @pl.when(i == 0)
        def _():
            gmax[...] = jnp.full((NUM_GRAPHS, C), -jnp.inf, jnp.float32)

        dinv = _dinv_of(deg_ref)
        acc = jnp.concatenate([a[0] + a[1] for a in accs], axis=1)
        hs = jnp.concatenate([h[...] for h in hss], axis=1)
        a5 = jnp.maximum(dinv * (acc + hs) + b4_ref[...], 0.0)
        bt = bt_ref[0]
        glo = jnp.min(bt)
        ghi = jnp.max(bt)

        def upd(g, carry):
            m = jnp.max(jnp.where(bt == g, a5, -jnp.inf), axis=0)
            cur = gmax[pl.ds(g, 1), :]
            gmax[pl.ds(g, 1), :] = jnp.maximum(cur, m[None, :])
            return carry

        lax.fori_loop(glo, ghi + 1, upd, 0)

        @pl.when(i == NBLK - 1)
        def _():
            gm = gmax[...]
            z = jnp.maximum(
                jnp.dot(gm, w5_ref[...],
                        preferred_element_type=jnp.float32,
                        precision=lax.Precision.HIGHEST) + b5_ref[...], 0.0)
            o_ref[...] = jnp.dot(
                z, w6_ref[...],
                preferred_element_type=jnp.float32,
                precision=lax.Precision.HIGHEST) + b6_ref[...]

    return pl.pallas_call(
        body,
        grid=(NBLK,),
        in_specs=[_deg_spec] + [_acc_spec] * P_in + [_hs_spec] * P_in + [
            _full_spec((1, C)),
            pl.BlockSpec((1, RBLK, 1), lambda i: (i, 0, 0)),
            _full_spec((NUM_GRAPHS, C)),
            _full_spec((1, C)),
            _full_spec((C, 10)),
            _full_spec((1, 10)),
        ],
        out_specs=_full_spec((NUM_GRAPHS, 10)),
        out_shape=jax.ShapeDtypeStruct((NUM_GRAPHS, 10), jnp.float32),
        scratch_shapes=[pltpu.VMEM((NUM_GRAPHS, C), jnp.float32)],
    )(deg, *acc_chunks, *hs_chunks, b4.reshape(1, C), batch3d,
      W5, b5.reshape(1, C), W6, b6.reshape(1, 10))


def kernel(x, edge_index, batch, W1, b1, W2, b2, W3, b3, W4, b4,
           W5, b5, W6, b6):
    pad = EPAD - E
    rowp = jnp.concatenate(
        [edge_index[0], jnp.zeros((pad,), edge_index.dtype)])
    colp = jnp.concatenate(
        [edge_index[1], jnp.full((pad,), N, edge_index.dtype)])
    row2d = rowp.reshape(EB, BATCH)
    col2d = colp.reshape(EB, BATCH)
    batch3d = batch.reshape(NBLK, RBLK, 1)

    deg = _sc_deg(col2d)
    hs = _tc_first(x, deg, W1)
    acc = _sc_prop(hs, row2d, col2d)
    hs = _tc_merge(deg, acc, hs, b1, W2)
    acc = _sc_prop(hs, row2d, col2d)
    hs = _tc_merge(deg, acc, hs, b2, W3)
    acc = _sc_prop(hs, row2d, col2d)
    hs = _tc_merge(deg, acc, hs, b3, W4)
    acc = _sc_prop(hs, row2d, col2d)
    return _tc_pool(deg, acc, hs, b4, batch3d, W5, b5, W6, b6)
